# baseline (speedup 1.0000x reference)
import functools

import jax
import jax.numpy as jnp
from jax import lax
from jax.experimental import pallas as pl
from jax.experimental.pallas import tpu as pltpu
from jax.experimental.pallas import tpu_sc as plsc

B = 4096
D = 64
Z = 10
NC = 2
NS = 16
NW = NC * NS
BPW = B // NW
NEG_CHUNK = 128
NEG_CHUNKS = B * Z // NEG_CHUNK
CPW = NEG_CHUNKS // NW


def _sc_gather_build():
    mesh = plsc.VectorSubcoreMesh(core_axis_name="c", subcore_axis_name="s")

    @functools.partial(
        pl.kernel,
        out_type=[
            jax.ShapeDtypeStruct((B, D), jnp.float32),
            jax.ShapeDtypeStruct((NEG_CHUNKS, NEG_CHUNK, D), jnp.float32),
        ],
        mesh=mesh,
        compiler_params=pltpu.CompilerParams(use_tc_tiling_on_sc=False),
        scratch_types=[
            pltpu.VMEM((BPW,), jnp.int32),
            pltpu.VMEM((CPW, NEG_CHUNK), jnp.int32),
            pltpu.VMEM((BPW, D), jnp.float32),
            pltpu.VMEM((CPW, NEG_CHUNK, D), jnp.float32),
            pltpu.SemaphoreType.DMA,
        ],
    )
    def sc_gather(doc_idx_hbm, neg_idx_hbm, table_hbm, pos_out, neg_out,
                  idx_pos_v, idx_neg_v, pos_rows_v, neg_rows_v, sem):
        w = lax.axis_index("s") * NC + lax.axis_index("c")
        pltpu.sync_copy(doc_idx_hbm.at[pl.ds(w * BPW, BPW)], idx_pos_v)
        pltpu.sync_copy(neg_idx_hbm.at[w], idx_neg_v)
        copies = [pltpu.async_copy(table_hbm.at[idx_pos_v], pos_rows_v, sem)]
        for j in range(CPW):
            copies.append(
                pltpu.async_copy(table_hbm.at[idx_neg_v.at[j]],
                                 neg_rows_v.at[j], sem))
        for c in copies:
            c.wait()
        pltpu.sync_copy(pos_rows_v, pos_out.at[pl.ds(w * BPW, BPW)])
        pltpu.sync_copy(neg_rows_v, neg_out.at[pl.ds(w * CPW, CPW)])

    return sc_gather


_sc_gather = _sc_gather_build()

BLK = 512


def _tc_score_body(q_ref, pos_ref, neg_ref, o_ref):
    q = q_ref[...]
    pos = pos_ref[...]
    neg = neg_ref[...]
    pos_dot = jnp.sum(pos * q, axis=1, keepdims=True)
    qt = jnp.concatenate([q] * Z, axis=1)
    prod = neg * qt
    r = lax.broadcasted_iota(jnp.int32, (Z * D, Z), 0)
    cc = lax.broadcasted_iota(jnp.int32, (Z * D, Z), 1)
    sel = (r // D == cc).astype(jnp.float32)
    neg_dot = lax.dot_general(prod, sel, (((1,), (0,)), ((), ())),
                              precision=lax.Precision.HIGHEST,
                              preferred_element_type=jnp.float32)
    pos_repr = 1.0 / (1.0 + jnp.exp(-pos_dot))
    neg_repr = 1.0 / (1.0 + jnp.exp(-neg_dot))
    positive_term = jnp.log(pos_repr)
    negative_term = jnp.sum(jnp.log(1.0 - neg_repr + 1e-40), axis=1,
                            keepdims=True)
    zf = float(Z)
    o_ref[...] = (zf + 1.0) / (2.0 * zf) * (zf * positive_term + negative_term)


def kernel(query, document, doc_emb, neg_sample):
    doc_i = document.astype(jnp.int32)
    neg_i = neg_sample.astype(jnp.int32).reshape(NW, CPW, NEG_CHUNK)
    pos_rows, neg_rows = _sc_gather(doc_i, neg_i, doc_emb)
    neg_flat = neg_rows.reshape(B, Z * D)
    out = pl.pallas_call(
        _tc_score_body,
        grid=(B // BLK,),
        in_specs=[
            pl.BlockSpec((BLK, D), lambda i: (i, 0)),
            pl.BlockSpec((BLK, D), lambda i: (i, 0)),
            pl.BlockSpec((BLK, Z * D), lambda i: (i, 0)),
        ],
        out_specs=pl.BlockSpec((BLK, 1), lambda i: (i, 0)),
        out_shape=jax.ShapeDtypeStruct((B, 1), jnp.float32),
    )(query, pos_rows, neg_flat)
    return out.reshape(B)


# speedup vs baseline: 1.5420x; 1.5420x over previous
import functools

import jax
import jax.numpy as jnp
from jax import lax
from jax.experimental import pallas as pl
from jax.experimental.pallas import tpu as pltpu
from jax.experimental.pallas import tpu_sc as plsc

B = 4096
D = 64
Z = 10
N_DOC = 1000000
NC = 2
NS = 16
NW = NC * NS
BPW = B // NW
EPW = BPW * (1 + Z)
CHUNK = 32
CHUNKS = EPW // CHUNK
POS_CHUNKS = BPW // CHUNK
SUBG = CHUNK // 16
PAIRS = CHUNKS // 2


def _sc_dots_build():
    mesh = plsc.VectorSubcoreMesh(core_axis_name="c", subcore_axis_name="s")

    @functools.partial(
        pl.kernel,
        out_type=jax.ShapeDtypeStruct((NW, EPW), jnp.float32),
        mesh=mesh,
        compiler_params=pltpu.CompilerParams(needs_layout_passes=False),
        scratch_types=[
            pltpu.VMEM((CHUNKS + 2, CHUNK), jnp.int32),
            pltpu.VMEM((BPW, D), jnp.float32),
            pltpu.VMEM((CHUNK, 8, D), jnp.float32),
            pltpu.VMEM((CHUNK, 8, D), jnp.float32),
            pltpu.VMEM((EPW,), jnp.float32),
            pltpu.SemaphoreType.DMA,
            pltpu.SemaphoreType.DMA,
        ],
    )
    def sc_dots(idx_hbm, q_hbm, table_hbm, out_hbm,
                idx_v, q_v, gbuf0, gbuf1, dots_v, sem0, sem1):
        w = lax.axis_index("s") * NC + lax.axis_index("c")
        pltpu.sync_copy(idx_hbm.at[w], idx_v)
        pltpu.sync_copy(q_hbm.at[pl.ds(w * BPW, BPW)], q_v)

        lane = lax.broadcasted_iota(jnp.int32, (16,), 0)

        def fire(c, gbuf, sem):
            for s in range(SUBG):
                gvec = idx_v[c, pl.ds(s * 16, 16)] >> 3
                for t in range(16):
                    pltpu.async_copy(table_hbm.at[gvec[t]],
                                     gbuf.at[s * 16 + t], sem)

        def drain(gbuf, sem):
            pltpu.make_async_copy(table_hbm.at[pl.ds(0, CHUNK)], gbuf,
                                  sem).wait()

        def compute(c, gbuf):
            def subgroup(s, _):
                raw = idx_v[c, pl.ds(s * 16, 16)]
                j_vec = lax.bitwise_and(raw, 7)
                r_vec = lane + s * 16
                base = c * CHUNK + s * 16 + lane
                neg_b = (base - POS_CHUNKS * CHUNK) // Z
                b_vec = jnp.where(jnp.full((16,), c < POS_CHUNKS), base,
                                  neg_b)

                def dstep(d, acc):
                    d_vec = jnp.full((16,), d, dtype=jnp.int32)
                    doc = plsc.load_gather(gbuf, [r_vec, j_vec, d_vec])
                    qv = plsc.load_gather(q_v, [b_vec, d_vec])
                    return acc + doc * qv

                acc = lax.fori_loop(0, D, dstep,
                                    jnp.zeros((16,), jnp.float32))
                dots_v[pl.ds(c * CHUNK + s * 16, 16)] = acc
                return 0

            lax.fori_loop(0, SUBG, subgroup, 0)

        fire(0, gbuf0, sem0)
        fire(1, gbuf1, sem1)

        def pair(i, _):
            c0 = 2 * i
            drain(gbuf0, sem0)
            compute(c0, gbuf0)
            fire(c0 + 2, gbuf0, sem0)
            drain(gbuf1, sem1)
            compute(c0 + 1, gbuf1)
            fire(c0 + 3, gbuf1, sem1)
            return 0

        lax.fori_loop(0, PAIRS, pair, 0)
        drain(gbuf0, sem0)
        drain(gbuf1, sem1)

        pltpu.sync_copy(dots_v, out_hbm.at[w])

    return sc_dots


_sc_dots = _sc_dots_build()


def _tc_score_body(pos_ref, neg_ref, o_ref):
    pos_dot = pos_ref[...]
    neg_dot = neg_ref[...]
    pos_repr = 1.0 / (1.0 + jnp.exp(-pos_dot))
    neg_repr = 1.0 / (1.0 + jnp.exp(-neg_dot))
    positive_term = jnp.log(pos_repr)
    negative_term = jnp.sum(jnp.log(1.0 - neg_repr + 1e-40), axis=1,
                            keepdims=True)
    zf = float(Z)
    o_ref[...] = (zf + 1.0) / (2.0 * zf) * (zf * positive_term + negative_term)


def kernel(query, document, doc_emb, neg_sample):
    doc_i = document.astype(jnp.int32).reshape(NW, BPW)
    neg_i = neg_sample.astype(jnp.int32).reshape(NW, BPW * Z)
    pad = jnp.zeros((NW, 2 * CHUNK), jnp.int32)
    idx_all = jnp.concatenate([doc_i, neg_i, pad], axis=1).reshape(
        NW, CHUNKS + 2, CHUNK)
    table3 = doc_emb.reshape(N_DOC // 8, 8, D)
    dots = _sc_dots(idx_all, query, table3)
    pos_dot = dots[:, :BPW].reshape(B, 1)
    neg_dot = dots[:, BPW:].reshape(B, Z)
    out = pl.pallas_call(
        _tc_score_body,
        out_shape=jax.ShapeDtypeStruct((B, 1), jnp.float32),
    )(pos_dot, neg_dot)
    return out.reshape(B)
